# speedup vs baseline: 1.1087x; 1.1087x over previous
import jax
import jax.numpy as jnp
from jax.experimental import pallas as pl


def _mm_body(x_ref, w_ref, o_ref):
    o_ref[...] = jnp.dot(x_ref[...], w_ref[...], preferred_element_type=jnp.float32)


def _matmul(x, W):
    M, _ = x.shape
    _, N = W.shape
    return pl.pallas_call(
        _mm_body,
        out_shape=jax.ShapeDtypeStruct((M, N), jnp.float32),
    )(x, W)


def _gat_layer(x, src, dst, W, a_src, a_dst, b, n_nodes):
    H, C = a_src.shape[1], a_src.shape[2]
    h = _matmul(x, W).reshape(n_nodes, H, C)
    alpha_src = (h * a_src).sum(axis=-1)
    alpha_dst = (h * a_dst).sum(axis=-1)
    alpha = jax.nn.leaky_relu(alpha_src[src] + alpha_dst[dst], negative_slope=0.2)
    amax = jax.ops.segment_max(alpha, dst, num_segments=n_nodes)
    amax = jnp.where(jnp.isfinite(amax), amax, 0.0)
    alpha = jnp.exp(alpha - jax.lax.stop_gradient(amax)[dst])
    denom = jax.ops.segment_sum(alpha, dst, num_segments=n_nodes)
    alpha = alpha / (denom[dst] + 1e-16)
    msg = h[src] * alpha[:, :, None]
    out = jax.ops.segment_sum(msg, dst, num_segments=n_nodes)
    return out.reshape(n_nodes, H * C) + b


def kernel(x, edge_index, W1, att_src1, att_dst1, b1, W2, att_src2, att_dst2, b2):
    n_nodes = x.shape[0]
    loops = jnp.arange(n_nodes, dtype=edge_index.dtype)
    ei = jnp.concatenate([edge_index, jnp.stack([loops, loops], axis=0)], axis=1)
    src, dst = ei[0], ei[1]
    h = _gat_layer(x, src, dst, W1, att_src1, att_dst1, b1, n_nodes)
    h = jax.nn.elu(h)
    out = _gat_layer(h, src, dst, W2, att_src2, att_dst2, b2, n_nodes)
    return out


# speedup vs baseline: 17.3537x; 15.6525x over previous
import dataclasses
import functools

import jax
import jax.numpy as jnp
from jax import lax
from jax.experimental import pallas as pl
from jax.experimental.pallas import tpu as pltpu
from jax.experimental.pallas import tpu_sc as plsc

N_NODES = 10000
E_REAL = 330000
K = 128
NSUB = 16
NCORE = 2
CH = 128
NCHUNK_A = 162
NCHUNK_B = 81
E_PAD = NCHUNK_A * K * NSUB


def _dense_body(split, x_ref, w_ref, asv_ref, adv_ref, h_ref, asrc_ref,
                adst_ref):
    h = jnp.dot(x_ref[...], w_ref[...], preferred_element_type=jnp.float32)
    if split:
        h_ref[0] = h[:, :CH]
        h_ref[1] = h[:, CH:]
    else:
        h_ref[...] = h
    asrc_ref[...] = jnp.sum(h * asv_ref[...], axis=1, keepdims=True)
    adst_ref[...] = jnp.sum(h * adv_ref[...], axis=1, keepdims=True)


def _dense_call(x, W, a_src, a_dst, split):
    n, _ = x.shape
    cout = W.shape[1]
    a_src = a_src.reshape(1, cout)
    a_dst = a_dst.reshape(1, cout)
    h_shape = (2, n, CH) if split else (n, cout)
    return pl.pallas_call(
        functools.partial(_dense_body, split),
        out_shape=[
            jax.ShapeDtypeStruct(h_shape, jnp.float32),
            jax.ShapeDtypeStruct((n, 1), jnp.float32),
            jax.ShapeDtypeStruct((n, 1), jnp.float32),
        ],
    )(x, W, a_src, a_dst)


def _edge_body(colsplit, h_hbm, src_hbm, dst_hbm, asrc_hbm, adst_hbm, mv_hbm,
               out_hbm, den_hbm,
               asrc_t, adst_t, srcb, dstb, sb2, eb, msg, mv, zline,
               acc, den_sh, gsem):
    c = lax.axis_index("c")
    s = lax.axis_index("s")
    nchunk = NCHUNK_A if colsplit else NCHUNK_B

    @pl.loop(0, K)
    def _(kk):
        for r in range(CH // 16):
            msg[0, kk, pl.ds(r * 16, 16)] = jnp.zeros((16,), jnp.float32)

    for r in range(640 // 16):
        zline[pl.ds(r * 16, 16)] = jnp.zeros((16,), jnp.float32)

    r0 = jnp.minimum(s * 640, N_NODES - 640)
    for t in range(5):
        pltpu.sync_copy(msg.at[0], acc.at[pl.ds(r0 + t * 128, 128)])

    @pl.when(s == 0)
    def _():
        @pl.loop(0, 15)
        def _(t):
            pltpu.sync_copy(zline, den_sh.at[pl.ds(t * 640, 640)])
        pltpu.sync_copy(zline.at[pl.ds(0, 400)], den_sh.at[pl.ds(9600, 400)])

    pltpu.sync_copy(asrc_hbm, asrc_t)
    pltpu.sync_copy(adst_hbm, adst_t)
    pltpu.sync_copy(mv_hbm, mv)
    plsc.subcore_barrier()

    mreg = mv[...]
    if colsplit:
        bias = c * N_NODES
        wstart = s * (NCHUNK_A * K)
    else:
        bias = 0
        wstart = (c * NSUB + s) * (NCHUNK_B * K)

    @pl.loop(0, nchunk)
    def _(g):
        base = wstart + g * K
        pltpu.sync_copy(src_hbm.at[pl.ds(base, K)], srcb.at[0])
        pltpu.sync_copy(dst_hbm.at[pl.ds(base, K)], dstb.at[0])
        for i in range(K // 16):
            sl = pl.ds(i * 16, 16)
            s16 = srcb[0, sl]
            d16 = dstb[0, sl]
            a_s = plsc.load_gather(asrc_t, [s16])
            a_d = plsc.load_gather(adst_t, [d16])
            z = a_s + a_d
            al = jnp.maximum(z, 0.2 * z) - mreg
            e16 = jnp.exp(al)
            eid = base + i * 16 + lax.iota(jnp.int32, 16)
            e16 = jnp.where(eid < E_REAL, e16, 0.0)
            eb[0, sl] = e16
            sb2[0, sl] = s16 + bias

        pltpu.async_copy(h_hbm.at[sb2.at[0]], msg.at[0], gsem).wait()

        @pl.loop(0, K)
        def _(kk):
            ev = plsc.load_gather(eb.at[0], [jnp.full((16,), kk, jnp.int32)])
            for r in range(CH // 16):
                sl = pl.ds(r * 16, 16)
                msg[0, kk, sl] = msg[0, kk, sl] * ev

        pltpu.sync_copy(msg.at[0], acc.at[dstb.at[0]], add=True)

        if colsplit:
            @pl.when(c == 0)
            def _():
                pltpu.sync_copy(eb.at[0], den_sh.at[dstb.at[0]], add=True)
        else:
            pltpu.sync_copy(eb.at[0], den_sh.at[dstb.at[0]], add=True)

    plsc.subcore_barrier()
    pltpu.sync_copy(acc.at[pl.ds(r0, 640)], out_hbm.at[c, pl.ds(r0, 640)])

    @pl.when(s == 0)
    def _():
        pltpu.sync_copy(den_sh, den_hbm.at[c])


def _edge_call(h2, src, dst, asrc, adst, mvec, colsplit):
    mesh = plsc.VectorSubcoreMesh(core_axis_name="c", subcore_axis_name="s")
    cp = pltpu.CompilerParams()
    if "needs_layout_passes" in pltpu.CompilerParams.__dataclass_fields__:
        cp = dataclasses.replace(cp, needs_layout_passes=False)
    kfn = pl.kernel(
        functools.partial(_edge_body, colsplit),
        compiler_params=cp,
        out_type=[
            jax.ShapeDtypeStruct((NCORE, N_NODES, CH), jnp.float32),
            jax.ShapeDtypeStruct((NCORE, N_NODES), jnp.float32),
        ],
        mesh=mesh,
        scratch_types=[
            pltpu.VMEM((N_NODES,), jnp.float32),
            pltpu.VMEM((N_NODES,), jnp.float32),
            pltpu.VMEM((1, K), jnp.int32),
            pltpu.VMEM((1, K), jnp.int32),
            pltpu.VMEM((1, K), jnp.int32),
            pltpu.VMEM((1, K), jnp.float32),
            pltpu.VMEM((1, K, CH), jnp.float32),
            pltpu.VMEM((16,), jnp.float32),
            pltpu.VMEM((640,), jnp.float32),
            pltpu.VMEM_SHARED((N_NODES, CH), jnp.float32),
            pltpu.VMEM_SHARED((N_NODES,), jnp.float32),
            pltpu.SemaphoreType.DMA,
        ],
    )
    return kfn(h2, src, dst, asrc, adst, mvec)


def _norm_body(colsplit, do_elu, u_ref, den_ref, b_ref, o_ref):
    inv = 1.0 / (den_ref[:, 0:1] + den_ref[:, 1:2] + 1e-16)
    if colsplit:
        y0 = u_ref[0] * inv + b_ref[:, :CH]
        y1 = u_ref[1] * inv + b_ref[:, CH:]
        if do_elu:
            y0 = jnp.where(y0 > 0, y0, jnp.exp(jnp.minimum(y0, 0.0)) - 1.0)
            y1 = jnp.where(y1 > 0, y1, jnp.exp(jnp.minimum(y1, 0.0)) - 1.0)
        o_ref[:, :CH] = y0
        o_ref[:, CH:] = y1
    else:
        y = (u_ref[0] + u_ref[1]) * inv + b_ref[...]
        if do_elu:
            y = jnp.where(y > 0, y, jnp.exp(jnp.minimum(y, 0.0)) - 1.0)
        o_ref[...] = y


def _norm_call(u, den, b, colsplit, do_elu):
    n = N_NODES
    cout = 2 * CH if colsplit else CH
    return pl.pallas_call(
        functools.partial(_norm_body, colsplit, do_elu),
        out_shape=jax.ShapeDtypeStruct((n, cout), jnp.float32),
    )(u, den.T, b.reshape(1, cout))


def _gat_layer(x, src, dst, W, a_src, a_dst, b, colsplit, do_elu):
    h2, asrc2, adst2 = _dense_call(x, W, a_src, a_dst, colsplit)
    asrc = asrc2.reshape(N_NODES)
    adst = adst2.reshape(N_NODES)
    ms = jnp.max(asrc) + jnp.max(adst)
    m = jnp.maximum(ms, 0.2 * ms)
    mvec = jnp.full((16,), m, jnp.float32)
    h_flat = h2.reshape(-1, CH)
    u, den = _edge_call(h_flat, src, dst, asrc, adst, mvec, colsplit)
    return _norm_call(u, den, b, colsplit, do_elu)


def kernel(x, edge_index, W1, att_src1, att_dst1, b1, W2, att_src2, att_dst2, b2):
    loops = jnp.arange(N_NODES, dtype=jnp.int32)
    src = jnp.concatenate([
        edge_index[0].astype(jnp.int32), loops,
        jnp.zeros((E_PAD - E_REAL,), jnp.int32)])
    dst = jnp.concatenate([
        edge_index[1].astype(jnp.int32), loops,
        jnp.zeros((E_PAD - E_REAL,), jnp.int32)])

    h = _gat_layer(x, src, dst, W1, att_src1, att_dst1, b1, True, True)
    out = _gat_layer(h, src, dst, W2, att_src2, att_dst2, b2, False, False)
    return out


# speedup vs baseline: 26.3287x; 1.5172x over previous
import dataclasses
import functools

import jax
import jax.numpy as jnp
from jax import lax
from jax.experimental import pallas as pl
from jax.experimental.pallas import tpu as pltpu
from jax.experimental.pallas import tpu_sc as plsc

N_NODES = 10000
E_REAL = 330000
K = 128
NSUB = 16
NCORE = 2
CH = 128
NCHUNK_A = 162
NCHUNK_B = 82
E_LEN = NCHUNK_B * K * NSUB * NCORE


def _dense_body(split, x_ref, w_ref, asv_ref, adv_ref, h_ref, asrc_ref,
                adst_ref):
    h = jnp.dot(x_ref[...], w_ref[...], preferred_element_type=jnp.float32)
    if split:
        h_ref[0] = h[:, :CH]
        h_ref[1] = h[:, CH:]
    else:
        h_ref[...] = h
    asrc_ref[...] = jnp.sum(h * asv_ref[...], axis=1, keepdims=True)
    adst_ref[...] = jnp.sum(h * adv_ref[...], axis=1, keepdims=True)


def _dense_call(x, W, a_src, a_dst, split):
    n, _ = x.shape
    cout = W.shape[1]
    a_src = a_src.reshape(1, cout)
    a_dst = a_dst.reshape(1, cout)
    h_shape = (2, n, CH) if split else (n, cout)
    return pl.pallas_call(
        functools.partial(_dense_body, split),
        out_shape=[
            jax.ShapeDtypeStruct(h_shape, jnp.float32),
            jax.ShapeDtypeStruct((n, 1), jnp.float32),
            jax.ShapeDtypeStruct((n, 1), jnp.float32),
        ],
    )(x, W, a_src, a_dst)


def _edge_body(colsplit, h_hbm, sd_hbm, asrc_hbm, adst_hbm, mv_hbm,
               out_hbm, den_hbm,
               sdb, sb2, eb, asb, adb, dsc, msg, zline,
               acc, den_sh,
               isem0, isem1, asem0, asem1, adsem0, adsem1,
               gsem0, gsem1, ssem0, ssem1, esem0, esem1):
    c = lax.axis_index("c")
    s = lax.axis_index("s")
    nchunk = NCHUNK_A if colsplit else NCHUNK_B
    isem = (isem0, isem1)
    asem = (asem0, asem1)
    adsem = (adsem0, adsem1)
    gsem = (gsem0, gsem1)
    ssem = (ssem0, ssem1)
    esem = (esem0, esem1)

    @pl.loop(0, K)
    def _(kk):
        for r in range(CH // 16):
            msg[0, kk, pl.ds(r * 16, 16)] = jnp.zeros((16,), jnp.float32)

    for r in range(640 // 16):
        zline[pl.ds(r * 16, 16)] = jnp.zeros((16,), jnp.float32)

    r0 = jnp.minimum(s * 640, N_NODES - 640)
    for t in range(5):
        pltpu.sync_copy(msg.at[0], acc.at[pl.ds(r0 + t * 128, 128)])

    @pl.when(s == 0)
    def _():
        @pl.loop(0, 15)
        def _(t):
            pltpu.sync_copy(zline, den_sh.at[pl.ds(t * 640, 640)])
        pltpu.sync_copy(zline.at[pl.ds(0, 400)], den_sh.at[pl.ds(9600, 400)])

    pltpu.sync_copy(mv_hbm, zline.at[pl.ds(0, 16)])
    mreg = zline[pl.ds(0, 16)]
    plsc.subcore_barrier()

    if colsplit:
        bias = c * N_NODES
        wstart = s * (NCHUNK_A * K)
    else:
        bias = 0
        wstart = (c * NSUB + s) * (NCHUNK_B * K)

    def wait_idx(b, nxt):
        pltpu.make_async_copy(sd_hbm.at[:, pl.ds(nxt, K)], sdb.at[b],
                              isem[b]).wait()

    def wait_logits(b):
        pltpu.make_async_copy(asrc_hbm.at[sdb.at[b, 0]], asb.at[b],
                              asem[b]).wait()
        pltpu.make_async_copy(adst_hbm.at[sdb.at[b, 1]], adb.at[b],
                              adsem[b]).wait()

    def wait_row_gather(b):
        pltpu.make_async_copy(h_hbm.at[sb2.at[b]], msg.at[b], gsem[b]).wait()

    def wait_row_scatter(b):
        pltpu.make_async_copy(msg.at[b], acc.at[dsc.at[b]], ssem[b]).wait()

    def issue_logit_gathers(b):
        pltpu.async_copy(asrc_hbm.at[sdb.at[b, 0]], asb.at[b], asem[b])
        pltpu.async_copy(adst_hbm.at[sdb.at[b, 1]], adb.at[b], adsem[b])

    def compute_sb2(b):
        for i in range(K // 16):
            sl = pl.ds(i * 16, 16)
            sb2[b, sl] = sdb[b, 0, sl] + bias

    def issue_row_gather(b):
        pltpu.async_copy(h_hbm.at[sb2.at[b]], msg.at[b], gsem[b])

    def compute_e(b, g):
        for i in range(K // 16):
            sl = pl.ds(i * 16, 16)
            z = asb[b, sl] + adb[b, sl]
            al = jnp.maximum(z, 0.2 * z) - mreg
            e16 = jnp.exp(al)
            eid = wstart + g * K + i * 16 + lax.iota(jnp.int32, 16)
            e16 = jnp.where(eid < E_REAL, e16, 0.0)
            eb[b, sl] = e16

    def multiply(b):
        @pl.loop(0, K, step=4)
        def _(k0):
            for u in range(4):
                kk = k0 + u
                ev = plsc.load_gather(eb.at[b],
                                      [jnp.full((16,), kk, jnp.int32)])
                for r in range(CH // 16):
                    sl = pl.ds(r * 16, 16)
                    msg[b, kk, sl] = msg[b, kk, sl] * ev

    def issue_scatters(b):
        for i in range(K // 16):
            sl = pl.ds(i * 16, 16)
            dsc[b, sl] = sdb[b, 1, sl]
        pltpu.async_copy(msg.at[b], acc.at[dsc.at[b]], ssem[b], add=True)
        if colsplit:
            @pl.when(c == 0)
            def _():
                pltpu.async_copy(eb.at[b], den_sh.at[dsc.at[b]],
                                 esem[b], add=True)
        else:
            pltpu.async_copy(eb.at[b], den_sh.at[dsc.at[b]],
                             esem[b], add=True)

    def wait_escatter(b):
        def w():
            pltpu.make_async_copy(eb.at[b], den_sh.at[dsc.at[b]],
                                  esem[b]).wait()
        if colsplit:
            @pl.when(c == 0)
            def _():
                w()
        else:
            w()

    pltpu.sync_copy(sd_hbm.at[:, pl.ds(wstart, K)], sdb.at[0])
    issue_logit_gathers(0)
    compute_sb2(0)
    issue_row_gather(0)

    @pl.loop(0, nchunk // 2)
    def _(gg):
        for b in (0, 1):
            g = gg * 2 + b
            nb = 1 - b
            nxt = wstart + (g + 1) * K

            def prep_next():
                wait_idx(nb, nxt)
                issue_logit_gathers(nb)
                compute_sb2(nb)
                if b == 0:
                    @pl.when(gg >= 1)
                    def _():
                        wait_row_scatter(nb)
                else:
                    wait_row_scatter(nb)
                issue_row_gather(nb)

            if b == 0:
                pltpu.async_copy(sd_hbm.at[:, pl.ds(nxt, K)], sdb.at[nb],
                                 isem[nb])
            else:
                @pl.when(g + 1 < nchunk)
                def _():
                    pltpu.async_copy(sd_hbm.at[:, pl.ds(nxt, K)], sdb.at[nb],
                                     isem[nb])

            wait_logits(b)

            @pl.when(gg >= 1)
            def _():
                wait_escatter(b)

            compute_e(b, g)

            if b == 0:
                prep_next()
            else:
                @pl.when(g + 1 < nchunk)
                def _():
                    prep_next()

            wait_row_gather(b)
            multiply(b)

            issue_scatters(b)

    for b in (0, 1):
        wait_row_scatter(b)
        wait_escatter(b)

    plsc.subcore_barrier()
    pltpu.sync_copy(acc.at[pl.ds(r0, 640)], out_hbm.at[c, pl.ds(r0, 640)])

    @pl.when(s == 0)
    def _():
        pltpu.sync_copy(den_sh, den_hbm.at[c])


def _edge_call(h2, sd, asrc, adst, mvec, colsplit):
    mesh = plsc.VectorSubcoreMesh(core_axis_name="c", subcore_axis_name="s")
    cp = pltpu.CompilerParams()
    if "needs_layout_passes" in pltpu.CompilerParams.__dataclass_fields__:
        cp = dataclasses.replace(cp, needs_layout_passes=False)
    kfn = pl.kernel(
        functools.partial(_edge_body, colsplit),
        compiler_params=cp,
        out_type=[
            jax.ShapeDtypeStruct((NCORE, N_NODES, CH), jnp.float32),
            jax.ShapeDtypeStruct((NCORE, N_NODES), jnp.float32),
        ],
        mesh=mesh,
        scratch_types=[
            pltpu.VMEM((2, 2, K), jnp.int32),
            pltpu.VMEM((2, K), jnp.int32),
            pltpu.VMEM((2, K), jnp.float32),
            pltpu.VMEM((2, K), jnp.float32),
            pltpu.VMEM((2, K), jnp.float32),
            pltpu.VMEM((2, K), jnp.int32),
            pltpu.VMEM((2, K, CH), jnp.float32),
            pltpu.VMEM((640,), jnp.float32),
            pltpu.VMEM_SHARED((N_NODES, CH), jnp.float32),
            pltpu.VMEM_SHARED((N_NODES,), jnp.float32),
        ] + [pltpu.SemaphoreType.DMA] * 12,
    )
    return kfn(h2, sd, asrc, adst, mvec)


def _norm_body(colsplit, do_elu, u_ref, den_ref, b_ref, o_ref):
    inv = 1.0 / (den_ref[:, 0:1] + den_ref[:, 1:2] + 1e-16)
    if colsplit:
        y0 = u_ref[0] * inv + b_ref[:, :CH]
        y1 = u_ref[1] * inv + b_ref[:, CH:]
        if do_elu:
            y0 = jnp.where(y0 > 0, y0, jnp.exp(jnp.minimum(y0, 0.0)) - 1.0)
            y1 = jnp.where(y1 > 0, y1, jnp.exp(jnp.minimum(y1, 0.0)) - 1.0)
        o_ref[:, :CH] = y0
        o_ref[:, CH:] = y1
    else:
        y = (u_ref[0] + u_ref[1]) * inv + b_ref[...]
        if do_elu:
            y = jnp.where(y > 0, y, jnp.exp(jnp.minimum(y, 0.0)) - 1.0)
        o_ref[...] = y


def _norm_call(u, den, b, colsplit, do_elu):
    n = N_NODES
    cout = 2 * CH if colsplit else CH
    return pl.pallas_call(
        functools.partial(_norm_body, colsplit, do_elu),
        out_shape=jax.ShapeDtypeStruct((n, cout), jnp.float32),
    )(u, den.T, b.reshape(1, cout))


def _gat_layer(x, sd, W, a_src, a_dst, b, colsplit, do_elu):
    h2, asrc2, adst2 = _dense_call(x, W, a_src, a_dst, colsplit)
    asrc = asrc2.reshape(N_NODES)
    adst = adst2.reshape(N_NODES)
    ms = jnp.max(asrc) + jnp.max(adst)
    m = jnp.maximum(ms, 0.2 * ms)
    mvec = jnp.full((16,), m, jnp.float32)
    h_flat = h2.reshape(-1, CH)
    u, den = _edge_call(h_flat, sd, asrc, adst, mvec, colsplit)
    return _norm_call(u, den, b, colsplit, do_elu)


def kernel(x, edge_index, W1, att_src1, att_dst1, b1, W2, att_src2, att_dst2, b2):
    loops = jnp.arange(N_NODES, dtype=jnp.int32)
    pad = jnp.zeros((2, E_LEN - E_REAL), jnp.int32)
    sd = jnp.concatenate(
        [edge_index.astype(jnp.int32),
         jnp.stack([loops, loops]), pad], axis=1)

    h = _gat_layer(x, sd, W1, att_src1, att_dst1, b1, True, True)
    out = _gat_layer(h, sd, W2, att_src2, att_dst2, b2, False, False)
    return out


# speedup vs baseline: 33.1252x; 1.2581x over previous
import dataclasses
import functools

import jax
import jax.numpy as jnp
from jax import lax
from jax.experimental import pallas as pl
from jax.experimental.pallas import tpu as pltpu
from jax.experimental.pallas import tpu_sc as plsc

N_NODES = 10000
E_REAL = 330000
K = 96
NSUB = 16
NCORE = 2
CH = 128
NCHUNK_A = 216
NCHUNK_B = 108
E_LEN = NCHUNK_A * K * NSUB
NCT = E_LEN // K
NBUF = 3


def _dense_body(split, x_ref, w_ref, asv_ref, adv_ref, h_ref, asrc_ref,
                adst_ref):
    h = jnp.dot(x_ref[...], w_ref[...], preferred_element_type=jnp.float32)
    if split:
        h_ref[0] = h[:, :CH]
        h_ref[1] = h[:, CH:]
    else:
        h_ref[...] = h
    asrc_ref[...] = jnp.sum(h * asv_ref[...], axis=1, keepdims=True)
    adst_ref[...] = jnp.sum(h * adv_ref[...], axis=1, keepdims=True)


def _dense_call(x, W, a_src, a_dst, split):
    n, _ = x.shape
    cout = W.shape[1]
    a_src = a_src.reshape(1, cout)
    a_dst = a_dst.reshape(1, cout)
    h_shape = (2, n, CH) if split else (n, cout)
    return pl.pallas_call(
        functools.partial(_dense_body, split),
        out_shape=[
            jax.ShapeDtypeStruct(h_shape, jnp.float32),
            jax.ShapeDtypeStruct((n, 1), jnp.float32),
            jax.ShapeDtypeStruct((n, 1), jnp.float32),
        ],
    )(x, W, a_src, a_dst)


def _edge_body(colsplit, h_hbm, sd_hbm, asrc_hbm, adst_hbm, mv_hbm,
               out_hbm, den_hbm,
               sdb, sb2, eb, asb, adb, dsc, msg, zline,
               acc, den_sh, *sems):
    c = lax.axis_index("c")
    s = lax.axis_index("s")
    nchunk = NCHUNK_A if colsplit else NCHUNK_B
    isem = sems[0:NBUF]
    asem = sems[NBUF:2 * NBUF]
    adsem = sems[2 * NBUF:3 * NBUF]
    gsem = sems[3 * NBUF:4 * NBUF]
    ssem = sems[4 * NBUF:5 * NBUF]
    esem = sems[5 * NBUF:6 * NBUF]

    @pl.loop(0, K)
    def _(kk):
        for r in range(CH // 16):
            msg[0, kk, pl.ds(r * 16, 16)] = jnp.zeros((16,), jnp.float32)

    for r in range(640 // 16):
        zline[pl.ds(r * 16, 16)] = jnp.zeros((16,), jnp.float32)

    r0 = jnp.minimum(s * 640, N_NODES - 640)
    for t in range(6):
        pltpu.sync_copy(msg.at[0], acc.at[pl.ds(r0 + t * 96, 96)])
    pltpu.sync_copy(msg.at[0, pl.ds(0, 64)], acc.at[pl.ds(r0 + 576, 64)])

    @pl.when(s == 0)
    def _():
        @pl.loop(0, 15)
        def _(t):
            pltpu.sync_copy(zline, den_sh.at[pl.ds(t * 640, 640)])
        pltpu.sync_copy(zline.at[pl.ds(0, 400)], den_sh.at[pl.ds(9600, 400)])

    pltpu.sync_copy(mv_hbm, zline.at[pl.ds(0, 16)])
    mreg = zline[pl.ds(0, 16)]
    plsc.subcore_barrier()

    if colsplit:
        bias = c * N_NODES
        wchunk = s * NCHUNK_A
    else:
        bias = 0
        wchunk = (c * NSUB + s) * NCHUNK_B
    wstart = wchunk * K

    def wait_idx(b, cid):
        pltpu.make_async_copy(sd_hbm.at[cid], sdb.at[b], isem[b]).wait()

    def wait_logits(b):
        pltpu.make_async_copy(asrc_hbm.at[sdb.at[b, 0]], asb.at[b],
                              asem[b]).wait()
        pltpu.make_async_copy(adst_hbm.at[sdb.at[b, 1]], adb.at[b],
                              adsem[b]).wait()

    def wait_row_gather(b):
        pltpu.make_async_copy(h_hbm.at[sb2.at[b]], msg.at[b], gsem[b]).wait()

    def wait_row_scatter(b):
        pltpu.make_async_copy(msg.at[b], acc.at[dsc.at[b]], ssem[b]).wait()

    def issue_logit_gathers(b):
        pltpu.async_copy(asrc_hbm.at[sdb.at[b, 0]], asb.at[b], asem[b])
        pltpu.async_copy(adst_hbm.at[sdb.at[b, 1]], adb.at[b], adsem[b])

    def compute_sb2(b):
        for i in range(K // 16):
            sl = pl.ds(i * 16, 16)
            sb2[b, sl] = sdb[b, 0, sl] + bias

    def issue_row_gather(b):
        pltpu.async_copy(h_hbm.at[sb2.at[b]], msg.at[b], gsem[b])

    def compute_e(b, g):
        for i in range(K // 16):
            sl = pl.ds(i * 16, 16)
            z = asb[b, sl] + adb[b, sl]
            al = jnp.maximum(z, 0.2 * z) - mreg
            e16 = jnp.exp(al)
            eid = wstart + g * K + i * 16 + lax.iota(jnp.int32, 16)
            e16 = jnp.where(eid < E_REAL, e16, 0.0)
            eb[b, sl] = e16

    def multiply(b):
        @pl.loop(0, K, step=4)
        def _(k0):
            for u in range(4):
                kk = k0 + u
                ev = plsc.load_gather(eb.at[b],
                                      [jnp.full((16,), kk, jnp.int32)])
                for r in range(CH // 16):
                    sl = pl.ds(r * 16, 16)
                    msg[b, kk, sl] = msg[b, kk, sl] * ev

    def issue_scatters(b):
        for i in range(K // 16):
            sl = pl.ds(i * 16, 16)
            dsc[b, sl] = sdb[b, 1, sl]
        pltpu.async_copy(msg.at[b], acc.at[dsc.at[b]], ssem[b], add=True)
        if colsplit:
            @pl.when(c == 0)
            def _():
                pltpu.async_copy(eb.at[b], den_sh.at[dsc.at[b]],
                                 esem[b], add=True)
        else:
            pltpu.async_copy(eb.at[b], den_sh.at[dsc.at[b]],
                             esem[b], add=True)

    def wait_escatter(b):
        def w():
            pltpu.make_async_copy(eb.at[b], den_sh.at[dsc.at[b]],
                                  esem[b]).wait()
        if colsplit:
            @pl.when(c == 0)
            def _():
                w()
        else:
            w()

    pltpu.sync_copy(sd_hbm.at[wchunk], sdb.at[0])
    pltpu.async_copy(sd_hbm.at[wchunk + 1], sdb.at[1], isem[1])
    issue_logit_gathers(0)
    compute_sb2(0)
    issue_row_gather(0)

    T = nchunk // NBUF

    @pl.loop(0, T)
    def _(t):
        for j in range(NBUF):
            g = t * NBUF + j
            p = j
            p1 = (j + 1) % NBUF
            p2 = (j + 2) % NBUF

            def issue_idx2():
                pltpu.async_copy(sd_hbm.at[wchunk + g + 2], sdb.at[p2],
                                 isem[p2])
            if j == 0:
                issue_idx2()
            else:
                @pl.when(t < T - 1)
                def _():
                    issue_idx2()

            wait_logits(p)

            @pl.when(t >= 1)
            def _():
                wait_escatter(p)

            compute_e(p, g)

            def prep_next():
                wait_idx(p1, wchunk + g + 1)
                issue_logit_gathers(p1)
                compute_sb2(p1)
                if j == 2:
                    wait_row_scatter(p1)
                else:
                    @pl.when(t >= 1)
                    def _():
                        wait_row_scatter(p1)
                issue_row_gather(p1)
            if j == 2:
                @pl.when(t < T - 1)
                def _():
                    prep_next()
            else:
                prep_next()

            wait_row_gather(p)
            multiply(p)

            issue_scatters(p)

    for b in range(NBUF):
        wait_row_scatter(b)
        wait_escatter(b)

    plsc.subcore_barrier()
    pltpu.sync_copy(acc.at[pl.ds(r0, 640)], out_hbm.at[c, pl.ds(r0, 640)])

    @pl.when(s == 0)
    def _():
        pltpu.sync_copy(den_sh, den_hbm.at[c])


def _edge_call(h2, sd, asrc, adst, mvec, colsplit):
    mesh = plsc.VectorSubcoreMesh(core_axis_name="c", subcore_axis_name="s")
    cp = pltpu.CompilerParams()
    if "needs_layout_passes" in pltpu.CompilerParams.__dataclass_fields__:
        cp = dataclasses.replace(cp, needs_layout_passes=False)
    kfn = pl.kernel(
        functools.partial(_edge_body, colsplit),
        compiler_params=cp,
        out_type=[
            jax.ShapeDtypeStruct((NCORE, N_NODES, CH), jnp.float32),
            jax.ShapeDtypeStruct((NCORE, N_NODES), jnp.float32),
        ],
        mesh=mesh,
        scratch_types=[
            pltpu.VMEM((NBUF, 2, K), jnp.int32),
            pltpu.VMEM((NBUF, K), jnp.int32),
            pltpu.VMEM((NBUF, K), jnp.float32),
            pltpu.VMEM((NBUF, K), jnp.float32),
            pltpu.VMEM((NBUF, K), jnp.float32),
            pltpu.VMEM((NBUF, K), jnp.int32),
            pltpu.VMEM((NBUF, K, CH), jnp.float32),
            pltpu.VMEM((640,), jnp.float32),
            pltpu.VMEM_SHARED((N_NODES, CH), jnp.float32),
            pltpu.VMEM_SHARED((N_NODES,), jnp.float32),
        ] + [pltpu.SemaphoreType.DMA] * (6 * NBUF),
    )
    return kfn(h2, sd, asrc, adst, mvec)


def _norm_body(colsplit, do_elu, u_ref, den_ref, b_ref, o_ref):
    inv = 1.0 / (den_ref[:, 0:1] + den_ref[:, 1:2] + 1e-16)
    if colsplit:
        y0 = u_ref[0] * inv + b_ref[:, :CH]
        y1 = u_ref[1] * inv + b_ref[:, CH:]
        if do_elu:
            y0 = jnp.where(y0 > 0, y0, jnp.exp(jnp.minimum(y0, 0.0)) - 1.0)
            y1 = jnp.where(y1 > 0, y1, jnp.exp(jnp.minimum(y1, 0.0)) - 1.0)
        o_ref[:, :CH] = y0
        o_ref[:, CH:] = y1
    else:
        y = (u_ref[0] + u_ref[1]) * inv + b_ref[...]
        if do_elu:
            y = jnp.where(y > 0, y, jnp.exp(jnp.minimum(y, 0.0)) - 1.0)
        o_ref[...] = y


def _norm_call(u, den, b, colsplit, do_elu):
    n = N_NODES
    cout = 2 * CH if colsplit else CH
    return pl.pallas_call(
        functools.partial(_norm_body, colsplit, do_elu),
        out_shape=jax.ShapeDtypeStruct((n, cout), jnp.float32),
    )(u, den.T, b.reshape(1, cout))


def _gat_layer(x, sd, W, a_src, a_dst, b, colsplit, do_elu):
    h2, asrc2, adst2 = _dense_call(x, W, a_src, a_dst, colsplit)
    asrc = asrc2.reshape(N_NODES)
    adst = adst2.reshape(N_NODES)
    ms = jnp.max(asrc) + jnp.max(adst)
    m = jnp.maximum(ms, 0.2 * ms)
    mvec = jnp.full((16,), m, jnp.float32)
    h_flat = h2.reshape(-1, CH)
    u, den = _edge_call(h_flat, sd, asrc, adst, mvec, colsplit)
    return _norm_call(u, den, b, colsplit, do_elu)


def kernel(x, edge_index, W1, att_src1, att_dst1, b1, W2, att_src2, att_dst2, b2):
    loops = jnp.arange(N_NODES, dtype=jnp.int32)
    pad = jnp.zeros((2, E_LEN - E_REAL), jnp.int32)
    sd = jnp.concatenate(
        [edge_index.astype(jnp.int32),
         jnp.stack([loops, loops]), pad], axis=1)
    sd = sd.reshape(2, NCT, K).transpose(1, 0, 2)

    h = _gat_layer(x, sd, W1, att_src1, att_dst1, b1, True, True)
    out = _gat_layer(h, sd, W2, att_src2, att_dst2, b2, False, False)
    return out


# speedup vs baseline: 33.7544x; 1.0190x over previous
import dataclasses
import functools

import jax
import jax.numpy as jnp
from jax import lax
from jax.experimental import pallas as pl
from jax.experimental.pallas import tpu as pltpu
from jax.experimental.pallas import tpu_sc as plsc

N_NODES = 10000
E_REAL = 330000
K = 96
NSUB = 16
NCORE = 2
CH = 128
NCHUNK_A = 216
NCHUNK_B = 108
E_LEN = NCHUNK_A * K * NSUB
NCT = E_LEN // K
NBUF = 3


def _dense_body(split, x_ref, w_ref, asv_ref, adv_ref, h_ref, asrc_ref,
                adst_ref, mv_ref):
    h = jnp.dot(x_ref[...], w_ref[...], preferred_element_type=jnp.float32)
    if split:
        h_ref[0] = h[:, :CH]
        h_ref[1] = h[:, CH:]
    else:
        h_ref[...] = h
    asrc = jnp.sum(h * asv_ref[...], axis=1, keepdims=True)
    adst = jnp.sum(h * adv_ref[...], axis=1, keepdims=True)
    asrc_ref[...] = asrc
    adst_ref[...] = adst
    ms = jnp.max(asrc) + jnp.max(adst)
    m = jnp.maximum(ms, 0.2 * ms)
    mv_ref[...] = jnp.full((1, 16), m, jnp.float32)


def _dense_call(x, W, a_src, a_dst, split):
    n, _ = x.shape
    cout = W.shape[1]
    a_src = a_src.reshape(1, cout)
    a_dst = a_dst.reshape(1, cout)
    h_shape = (2, n, CH) if split else (n, cout)
    return pl.pallas_call(
        functools.partial(_dense_body, split),
        out_shape=[
            jax.ShapeDtypeStruct(h_shape, jnp.float32),
            jax.ShapeDtypeStruct((n, 1), jnp.float32),
            jax.ShapeDtypeStruct((n, 1), jnp.float32),
            jax.ShapeDtypeStruct((1, 16), jnp.float32),
        ],
    )(x, W, a_src, a_dst)


def _edge_body(colsplit, h_hbm, sd_hbm, asrc_hbm, adst_hbm, mv_hbm,
               out_hbm, den_hbm,
               sdb, sb2, eb, asb, adb, dsc, msg, zline,
               acc, den_sh, *sems):
    c = lax.axis_index("c")
    s = lax.axis_index("s")
    nchunk = NCHUNK_A if colsplit else NCHUNK_B
    isem = sems[0:NBUF]
    asem = sems[NBUF:2 * NBUF]
    adsem = sems[2 * NBUF:3 * NBUF]
    gsem = sems[3 * NBUF:4 * NBUF]
    ssem = sems[4 * NBUF:5 * NBUF]
    esem = sems[5 * NBUF:6 * NBUF]

    @pl.loop(0, K)
    def _(kk):
        for r in range(CH // 16):
            msg[0, kk, pl.ds(r * 16, 16)] = jnp.zeros((16,), jnp.float32)

    for r in range(640 // 16):
        zline[pl.ds(r * 16, 16)] = jnp.zeros((16,), jnp.float32)

    r0 = jnp.minimum(s * 640, N_NODES - 640)
    for t in range(6):
        pltpu.sync_copy(msg.at[0], acc.at[pl.ds(r0 + t * 96, 96)])
    pltpu.sync_copy(msg.at[0, pl.ds(0, 64)], acc.at[pl.ds(r0 + 576, 64)])

    @pl.when(s == 0)
    def _():
        @pl.loop(0, 15)
        def _(t):
            pltpu.sync_copy(zline, den_sh.at[pl.ds(t * 640, 640)])
        pltpu.sync_copy(zline.at[pl.ds(0, 400)], den_sh.at[pl.ds(9600, 400)])

    pltpu.sync_copy(mv_hbm, zline.at[pl.ds(0, 16)])
    mreg = zline[pl.ds(0, 16)]
    plsc.subcore_barrier()

    if colsplit:
        bias = c * N_NODES
        wchunk = s * NCHUNK_A
    else:
        bias = 0
        wchunk = (s * NCORE + c) * NCHUNK_B
    wstart = wchunk * K

    def wait_idx(b, cid):
        pltpu.make_async_copy(sd_hbm.at[cid], sdb.at[b], isem[b]).wait()

    def wait_logits(b):
        pltpu.make_async_copy(asrc_hbm.at[sdb.at[b, 0]], asb.at[b],
                              asem[b]).wait()
        pltpu.make_async_copy(adst_hbm.at[sdb.at[b, 1]], adb.at[b],
                              adsem[b]).wait()

    def wait_row_gather(b):
        pltpu.make_async_copy(h_hbm.at[sb2.at[b]], msg.at[b], gsem[b]).wait()

    def wait_row_scatter(b):
        pltpu.make_async_copy(msg.at[b], acc.at[dsc.at[b]], ssem[b]).wait()

    def issue_logit_gathers(b):
        pltpu.async_copy(asrc_hbm.at[sdb.at[b, 0]], asb.at[b], asem[b])
        pltpu.async_copy(adst_hbm.at[sdb.at[b, 1]], adb.at[b], adsem[b])

    def compute_sb2(b):
        for i in range(K // 16):
            sl = pl.ds(i * 16, 16)
            sb2[b, sl] = sdb[b, 0, sl] + bias

    def issue_row_gather(b):
        pltpu.async_copy(h_hbm.at[sb2.at[b]], msg.at[b], gsem[b])

    def compute_e(b, g):
        for i in range(K // 16):
            sl = pl.ds(i * 16, 16)
            z = asb[b, sl] + adb[b, sl]
            al = jnp.maximum(z, 0.2 * z) - mreg
            e16 = jnp.exp(al)
            eid = wstart + g * K + i * 16 + lax.iota(jnp.int32, 16)
            e16 = jnp.where(eid < E_REAL, e16, 0.0)
            eb[b, sl] = e16

    def multiply(b):
        @pl.loop(0, K, step=4)
        def _(k0):
            for u in range(4):
                kk = k0 + u
                ev = plsc.load_gather(eb.at[b],
                                      [jnp.full((16,), kk, jnp.int32)])
                for r in range(CH // 16):
                    sl = pl.ds(r * 16, 16)
                    msg[b, kk, sl] = msg[b, kk, sl] * ev

    def issue_scatters(b):
        for i in range(K // 16):
            sl = pl.ds(i * 16, 16)
            dsc[b, sl] = sdb[b, 1, sl]
        pltpu.async_copy(msg.at[b], acc.at[dsc.at[b]], ssem[b], add=True)
        if colsplit:
            @pl.when(c == 0)
            def _():
                pltpu.async_copy(eb.at[b], den_sh.at[dsc.at[b]],
                                 esem[b], add=True)
        else:
            pltpu.async_copy(eb.at[b], den_sh.at[dsc.at[b]],
                             esem[b], add=True)

    def wait_escatter(b):
        def w():
            pltpu.make_async_copy(eb.at[b], den_sh.at[dsc.at[b]],
                                  esem[b]).wait()
        if colsplit:
            @pl.when(c == 0)
            def _():
                w()
        else:
            w()

    pltpu.sync_copy(sd_hbm.at[wchunk], sdb.at[0])
    pltpu.async_copy(sd_hbm.at[wchunk + 1], sdb.at[1], isem[1])
    issue_logit_gathers(0)
    compute_sb2(0)
    issue_row_gather(0)

    T = nchunk // NBUF

    @pl.loop(0, T)
    def _(t):
        for j in range(NBUF):
            g = t * NBUF + j
            p = j
            p1 = (j + 1) % NBUF
            p2 = (j + 2) % NBUF

            def issue_idx2():
                pltpu.async_copy(sd_hbm.at[wchunk + g + 2], sdb.at[p2],
                                 isem[p2])
            if j == 0:
                issue_idx2()
            else:
                @pl.when(t < T - 1)
                def _():
                    issue_idx2()

            wait_logits(p)

            @pl.when(t >= 1)
            def _():
                wait_escatter(p)

            compute_e(p, g)

            def prep_next():
                wait_idx(p1, wchunk + g + 1)
                issue_logit_gathers(p1)
                compute_sb2(p1)
                if j == 2:
                    wait_row_scatter(p1)
                else:
                    @pl.when(t >= 1)
                    def _():
                        wait_row_scatter(p1)
                issue_row_gather(p1)
            if j == 2:
                @pl.when(t < T - 1)
                def _():
                    prep_next()
            else:
                prep_next()

            wait_row_gather(p)
            multiply(p)

            issue_scatters(p)

    for b in range(NBUF):
        wait_row_scatter(b)
        wait_escatter(b)

    plsc.subcore_barrier()
    pltpu.sync_copy(acc.at[pl.ds(r0, 640)], out_hbm.at[c, pl.ds(r0, 640)])

    @pl.when(s == 0)
    def _():
        pltpu.sync_copy(den_sh, den_hbm.at[c])


def _edge_call(h2, sd, asrc, adst, mvec, colsplit):
    mesh = plsc.VectorSubcoreMesh(core_axis_name="c", subcore_axis_name="s")
    cp = pltpu.CompilerParams()
    if "needs_layout_passes" in pltpu.CompilerParams.__dataclass_fields__:
        cp = dataclasses.replace(cp, needs_layout_passes=False)
    kfn = pl.kernel(
        functools.partial(_edge_body, colsplit),
        compiler_params=cp,
        out_type=[
            jax.ShapeDtypeStruct((NCORE, N_NODES, CH), jnp.float32),
            jax.ShapeDtypeStruct((NCORE, N_NODES), jnp.float32),
        ],
        mesh=mesh,
        scratch_types=[
            pltpu.VMEM((NBUF, 2, K), jnp.int32),
            pltpu.VMEM((NBUF, K), jnp.int32),
            pltpu.VMEM((NBUF, K), jnp.float32),
            pltpu.VMEM((NBUF, K), jnp.float32),
            pltpu.VMEM((NBUF, K), jnp.float32),
            pltpu.VMEM((NBUF, K), jnp.int32),
            pltpu.VMEM((NBUF, K, CH), jnp.float32),
            pltpu.VMEM((640,), jnp.float32),
            pltpu.VMEM_SHARED((N_NODES, CH), jnp.float32),
            pltpu.VMEM_SHARED((N_NODES,), jnp.float32),
        ] + [pltpu.SemaphoreType.DMA] * (6 * NBUF),
    )
    return kfn(h2, sd, asrc, adst, mvec)


def _norm_body(colsplit, do_elu, u_ref, den_ref, b_ref, o_ref):
    inv = 1.0 / (den_ref[:, 0:1] + den_ref[:, 1:2] + 1e-16)
    if colsplit:
        y0 = u_ref[0] * inv + b_ref[:, :CH]
        y1 = u_ref[1] * inv + b_ref[:, CH:]
        if do_elu:
            y0 = jnp.where(y0 > 0, y0, jnp.exp(jnp.minimum(y0, 0.0)) - 1.0)
            y1 = jnp.where(y1 > 0, y1, jnp.exp(jnp.minimum(y1, 0.0)) - 1.0)
        o_ref[:, :CH] = y0
        o_ref[:, CH:] = y1
    else:
        y = (u_ref[0] + u_ref[1]) * inv + b_ref[...]
        if do_elu:
            y = jnp.where(y > 0, y, jnp.exp(jnp.minimum(y, 0.0)) - 1.0)
        o_ref[...] = y


def _norm_call(u, den, b, colsplit, do_elu):
    n = N_NODES
    cout = 2 * CH if colsplit else CH
    return pl.pallas_call(
        functools.partial(_norm_body, colsplit, do_elu),
        out_shape=jax.ShapeDtypeStruct((n, cout), jnp.float32),
    )(u, den.T, b.reshape(1, cout))


def _gat_layer(x, sd, W, a_src, a_dst, b, colsplit, do_elu):
    h2, asrc2, adst2, mv2 = _dense_call(x, W, a_src, a_dst, colsplit)
    asrc = asrc2.reshape(N_NODES)
    adst = adst2.reshape(N_NODES)
    mvec = mv2.reshape(16)
    h_flat = h2.reshape(-1, CH)
    u, den = _edge_call(h_flat, sd, asrc, adst, mvec, colsplit)
    return _norm_call(u, den, b, colsplit, do_elu)


def kernel(x, edge_index, W1, att_src1, att_dst1, b1, W2, att_src2, att_dst2, b2):
    loops = jnp.arange(N_NODES, dtype=jnp.int32)
    pad = jnp.zeros((2, E_LEN - E_REAL), jnp.int32)
    sd = jnp.concatenate(
        [edge_index.astype(jnp.int32),
         jnp.stack([loops, loops]), pad], axis=1)
    sd = sd.reshape(2, NCT, K).transpose(1, 0, 2)

    h = _gat_layer(x, sd, W1, att_src1, att_dst1, b1, True, True)
    out = _gat_layer(h, sd, W2, att_src2, att_dst2, b2, False, False)
    return out
